# baseline (speedup 1.0000x reference)
import functools

import jax
import jax.numpy as jnp
from jax import lax
from jax.experimental import pallas as pl
from jax.experimental.pallas import tpu as pltpu
from jax.experimental.pallas import tpu_sc as plsc

D = 64
SCALE = 8.0
G = 128
K = 8
NW = 32
LANES = 16


def _make_gather(B):
    assert B % (NW * G * K) == 0
    b_per_w = B // NW
    ng = b_per_w // G
    n_super = ng // K
    mesh = plsc.VectorSubcoreMesh(core_axis_name="c", subcore_axis_name="s")

    @functools.partial(
        pl.kernel,
        mesh=mesh,
        out_type=jax.ShapeDtypeStruct((B, D), jnp.float32),
        scratch_types=[
            pltpu.VMEM((ng, G), jnp.int32),
            pltpu.VMEM((K * G, D), jnp.float32),
            pltpu.SemaphoreType.DMA,
        ],
        compiler_params=pltpu.CompilerParams(use_tc_tiling_on_sc=False),
    )
    def gather_scale(x_hbm, table_hbm, out_hbm, idx_v, rows_v, sem):
        wid = lax.axis_index("s") * 2 + lax.axis_index("c")
        row_base = wid * ng
        out_base = wid * b_per_w

        pltpu.sync_copy(x_hbm.at[pl.ds(row_base, ng)], idx_v)

        def super_group(sg, carry):
            copies = []
            for j in range(K):
                copies.append(
                    pltpu.async_copy(
                        table_hbm.at[idx_v.at[sg * K + j]],
                        rows_v.at[pl.ds(j * G, G)],
                        sem,
                    )
                )
            for c in copies:
                c.wait()

            def scale_row(r, rc):
                for c in range(D // LANES):
                    sl = pl.ds(c * LANES, LANES)
                    rows_v[r, sl] = rows_v[r, sl] * SCALE
                return rc

            lax.fori_loop(0, K * G, scale_row, 0, unroll=4)

            pltpu.sync_copy(
                rows_v, out_hbm.at[pl.ds(out_base + sg * (K * G), K * G)]
            )
            return carry

        lax.fori_loop(0, n_super, super_group, 0)

    return gather_scale


@jax.jit
def kernel(x, table):
    B = x.size
    xf = x.astype(jnp.int32).reshape(B // G, G)
    out = _make_gather(B)(xf, table)
    return out.reshape(*x.shape, D)


# speedup vs baseline: 1.0489x; 1.0489x over previous
import functools

import jax
import jax.numpy as jnp
from jax import lax
from jax.experimental import pallas as pl
from jax.experimental.pallas import tpu as pltpu
from jax.experimental.pallas import tpu_sc as plsc

D = 64
SCALE = 8.0
G = 128
K = 2
NB = 4
NW = 32
LANES = 16


def _make_gather(B):
    assert B % (NW * G * K * NB) == 0
    b_per_w = B // NW
    ng = b_per_w // G
    n_grp = ng // K
    n_outer = n_grp // NB
    GR = K * G
    mesh = plsc.VectorSubcoreMesh(core_axis_name="c", subcore_axis_name="s")

    @functools.partial(
        pl.kernel,
        mesh=mesh,
        out_type=jax.ShapeDtypeStruct((B, D), jnp.float32),
        scratch_types=[
            pltpu.VMEM((ng, G), jnp.int32),
            pltpu.VMEM((NB, GR, D), jnp.float32),
        ]
        + [pltpu.SemaphoreType.DMA] * NB
        + [pltpu.SemaphoreType.DMA] * NB,
        compiler_params=pltpu.CompilerParams(use_tc_tiling_on_sc=False),
    )
    def gather_scale(x_hbm, table_hbm, out_hbm, idx_v, rows_v, *sems):
        gsem = sems[:NB]
        wsem = sems[NB:]
        wid = lax.axis_index("s") * 2 + lax.axis_index("c")
        row_base = wid * ng
        out_base = wid * b_per_w

        pltpu.sync_copy(x_hbm.at[pl.ds(row_base, ng)], idx_v)

        def fire_gathers(grp, buf):
            for j in range(K):
                pltpu.async_copy(
                    table_hbm.at[idx_v.at[grp * K + j]],
                    rows_v.at[buf, pl.ds(j * G, G)],
                    gsem[buf],
                )

        def wait_gathers(buf):
            for j in range(K):
                pltpu.make_async_copy(
                    table_hbm.at[idx_v.at[0]],
                    rows_v.at[buf, pl.ds(j * G, G)],
                    gsem[buf],
                ).wait()

        def scale_buf(buf):
            def scale_row(r, rc):
                for c in range(D // LANES):
                    sl = pl.ds(c * LANES, LANES)
                    rows_v[buf, r, sl] = rows_v[buf, r, sl] * SCALE
                return rc

            lax.fori_loop(0, GR, scale_row, 0, unroll=8)

        def fire_write(grp, buf):
            pltpu.async_copy(
                rows_v.at[buf], out_hbm.at[pl.ds(out_base + grp * GR, GR)],
                wsem[buf],
            )

        def wait_write(buf):
            pltpu.make_async_copy(
                rows_v.at[buf], out_hbm.at[pl.ds(out_base, GR)], wsem[buf]
            ).wait()

        for b in range(NB - 1):
            fire_gathers(b, b)

        def outer(t, carry):
            for b in range(NB):
                g = t * NB + b
                nbuf = (b - 1) % NB

                @pl.when(g + NB - 1 < n_grp)
                def _():
                    @pl.when(g > 0)
                    def _():
                        wait_write(nbuf)

                    fire_gathers(g + NB - 1, nbuf)

                wait_gathers(b)
                scale_buf(b)
                fire_write(g, b)
            return carry

        lax.fori_loop(0, n_outer, outer, 0)

        for b in range(NB):
            wait_write(b)

    return gather_scale


@jax.jit
def kernel(x, table):
    B = x.size
    xf = x.astype(jnp.int32).reshape(B // G, G)
    out = _make_gather(B)(xf, table)
    return out.reshape(*x.shape, D)
